# baseline (speedup 1.0000x reference)
import functools

import jax
import jax.numpy as jnp
from jax import lax
from jax.experimental import pallas as pl
from jax.experimental.pallas import tpu as pltpu
from jax.experimental.pallas import tpu_sc as plsc

N = 10000
NP = 10240
E = 320000
D = 128
H = 128
NC = 2
NS = 16
NW = NC * NS
EPW = E // NW
BB = 80
NB = EPW // BB
RPS = NP // NS
RB = 1024
LEAK = 0.2


def _tc_in(x_pad, W, asrc, adst):
    def body(x_ref, w_ref, as_ref, ad_ref, z_ref, elr_ref, elm_ref):
        z = jnp.dot(x_ref[...], w_ref[...], preferred_element_type=jnp.float32)
        z_ref[...] = z
        el = jnp.sum(z * as_ref[...], axis=1)
        er = jnp.sum(z * ad_ref[...], axis=1)
        elr_ref[...] = jnp.stack([el, er], axis=0)
        @pl.when(pl.program_id(0) == 0)
        def _():
            elm_ref[...] = jnp.full((1, H), -3e38, jnp.float32)
        elm_ref[...] = jnp.maximum(elm_ref[...], jnp.max(el))

    return pl.pallas_call(
        body,
        grid=(NP // RB,),
        in_specs=[
            pl.BlockSpec((RB, D), lambda i: (i, 0)),
            pl.BlockSpec((D, H), lambda i: (0, 0)),
            pl.BlockSpec((1, H), lambda i: (0, 0)),
            pl.BlockSpec((1, H), lambda i: (0, 0)),
        ],
        out_specs=[
            pl.BlockSpec((RB, H), lambda i: (i, 0)),
            pl.BlockSpec((2, RB), lambda i: (0, i)),
            pl.BlockSpec((1, H), lambda i: (0, 0)),
        ],
        out_shape=[
            jax.ShapeDtypeStruct((NP, H), jnp.float32),
            jax.ShapeDtypeStruct((2, NP), jnp.float32),
            jax.ShapeDtypeStruct((1, H), jnp.float32),
        ],
    )(x_pad, W, asrc, adst)


def _tc_mid(part, dens, b, W, asrc, adst):
    def body(p_ref, dn_ref, b_ref, w_ref, as_ref, ad_ref, z_ref, elr_ref,
             elm_ref):
        dt = jnp.sum(dn_ref[...], axis=0) + 1e-38
        agg = (p_ref[0] + p_ref[1]) / dt[:, None] + b_ref[...]
        h = jnp.maximum(agg, 0.0)
        z = jnp.dot(h, w_ref[...], preferred_element_type=jnp.float32)
        z_ref[...] = z
        el = jnp.sum(z * as_ref[...], axis=1)
        er = jnp.sum(z * ad_ref[...], axis=1)
        elr_ref[...] = jnp.stack([el, er], axis=0)
        @pl.when(pl.program_id(0) == 0)
        def _():
            elm_ref[...] = jnp.full((1, H), -3e38, jnp.float32)
        elm_ref[...] = jnp.maximum(elm_ref[...], jnp.max(el))

    return pl.pallas_call(
        body,
        grid=(NP // RB,),
        in_specs=[
            pl.BlockSpec((2, RB, H), lambda i: (0, i, 0)),
            pl.BlockSpec((NW, RB), lambda i: (0, i)),
            pl.BlockSpec((1, H), lambda i: (0, 0)),
            pl.BlockSpec((D, H), lambda i: (0, 0)),
            pl.BlockSpec((1, H), lambda i: (0, 0)),
            pl.BlockSpec((1, H), lambda i: (0, 0)),
        ],
        out_specs=[
            pl.BlockSpec((RB, H), lambda i: (i, 0)),
            pl.BlockSpec((2, RB), lambda i: (0, i)),
            pl.BlockSpec((1, H), lambda i: (0, 0)),
        ],
        out_shape=[
            jax.ShapeDtypeStruct((NP, H), jnp.float32),
            jax.ShapeDtypeStruct((2, NP), jnp.float32),
            jax.ShapeDtypeStruct((1, H), jnp.float32),
        ],
    )(part, dens, b, W, asrc, adst)


def _tc_out(part, dens, b):
    def body(p_ref, dn_ref, b_ref, o_ref):
        dt = jnp.sum(dn_ref[...], axis=0) + 1e-38
        o_ref[...] = (p_ref[0] + p_ref[1]) / dt[:, None] + b_ref[...]

    return pl.pallas_call(
        body,
        grid=(NP // RB,),
        in_specs=[
            pl.BlockSpec((2, RB, H), lambda i: (0, i, 0)),
            pl.BlockSpec((NW, RB), lambda i: (0, i)),
            pl.BlockSpec((1, H), lambda i: (0, 0)),
        ],
        out_specs=pl.BlockSpec((RB, H), lambda i: (i, 0)),
        out_shape=jax.ShapeDtypeStruct((NP, H), jnp.float32),
    )(part, dens, b)


def _sc_att(el, er, elmax, srcf, dstf, zer1d):
    mesh = plsc.VectorSubcoreMesh(core_axis_name="c", subcore_axis_name="s")

    @functools.partial(
        pl.kernel,
        out_type=[
            jax.ShapeDtypeStruct((E,), jnp.float32),
            jax.ShapeDtypeStruct((NW, NP), jnp.float32),
        ],
        mesh=mesh,
        compiler_params=pltpu.CompilerParams(needs_layout_passes=False),
        scratch_types=[
            pltpu.VMEM((EPW,), jnp.int32),
            pltpu.VMEM((EPW,), jnp.int32),
            pltpu.VMEM((NP,), jnp.float32),
            pltpu.VMEM((NP,), jnp.float32),
            pltpu.VMEM((NP,), jnp.float32),
            pltpu.VMEM((1, H), jnp.float32),
            pltpu.VMEM((EPW,), jnp.float32),
        ],
    )
    def k(el_hbm, er_hbm, elm_hbm, srcf_hbm, dstf_hbm, z1_hbm,
          eh_hbm, den_hbm,
          src_v, dst_v, el_v, er_v, den_v, elm_v, eh_v):
        c = lax.axis_index("c")
        s = lax.axis_index("s")
        wid = s * NC + c

        pltpu.sync_copy(srcf_hbm.at[pl.ds(wid * EPW, EPW)], src_v)
        pltpu.sync_copy(dstf_hbm.at[pl.ds(wid * EPW, EPW)], dst_v)
        pltpu.sync_copy(el_hbm, el_v)
        pltpu.sync_copy(er_hbm, er_v)
        pltpu.sync_copy(elm_hbm, elm_v)
        pltpu.sync_copy(z1_hbm, den_v)

        elm = elm_v[0, pl.ds(0, 16)]

        def grp(g, carry):
            sl = pl.ds(g * 16, 16)
            sv = src_v[sl]
            dv = dst_v[sl]
            els = plsc.load_gather(el_v, [sv])
            erd = plsc.load_gather(er_v, [dv])
            e = els + erd
            e = jnp.where(e > 0.0, e, LEAK * e)
            t = erd + elm
            ub = jnp.where(t > 0.0, t, LEAK * t)
            eh = jnp.exp(e - ub)
            eh_v[sl] = eh
            plsc.addupdate_scatter(den_v, [dv], eh)
            return carry

        lax.fori_loop(0, EPW // 16, grp, 0)

        pltpu.sync_copy(eh_v, eh_hbm.at[pl.ds(wid * EPW, EPW)])
        pltpu.sync_copy(den_v, den_hbm.at[wid])

    return k(el, er, elmax, srcf, dstf, zer1d)


def _sc_agg(z_pad, eh, srcf, dstg, zer2d):
    mesh = plsc.VectorSubcoreMesh(core_axis_name="c", subcore_axis_name="s")

    @functools.partial(
        pl.kernel,
        out_type=jax.ShapeDtypeStruct((NC, NP, H), jnp.float32),
        mesh=mesh,
        compiler_params=pltpu.CompilerParams(needs_layout_passes=False),
        scratch_types=[
            pltpu.VMEM((EPW,), jnp.int32),
            pltpu.VMEM((3, BB), jnp.int32),
            pltpu.VMEM((3, BB), jnp.float32),
            pltpu.VMEM((3, BB, H), jnp.float32),
            pltpu.VMEM_SHARED((NP, H), jnp.float32),
            pltpu.SemaphoreType.DMA,
            pltpu.SemaphoreType.DMA,
            pltpu.SemaphoreType.DMA,
            pltpu.SemaphoreType.DMA,
            pltpu.SemaphoreType.DMA,
            pltpu.SemaphoreType.DMA,
        ],
    )
    def k(z_hbm, eh_hbm, srcf_hbm, dstg_hbm, z2_hbm,
          part_hbm,
          src_v, dst2_v, eh_v, rows_v, acc,
          sem0, sem1, sem2, ssem0, ssem1, ssem2):
        c = lax.axis_index("c")
        s = lax.axis_index("s")
        wid = s * NC + c

        pltpu.sync_copy(srcf_hbm.at[pl.ds(wid * EPW, EPW)], src_v)
        pltpu.sync_copy(z2_hbm, acc.at[pl.ds(s * RPS, RPS), :])

        plsc.subcore_barrier()

        sems = (sem0, sem1, sem2)
        ssems = (ssem0, ssem1, ssem2)

        def start(b, buf):
            sem = sems[buf]
            pltpu.async_copy(
                z_hbm.at[src_v.at[pl.ds(b * BB, BB)]],
                rows_v.at[buf], sem)
            pltpu.async_copy(dstg_hbm.at[wid, b], dst2_v.at[buf], sem)
            pltpu.async_copy(
                eh_hbm.at[pl.ds(wid * EPW + b * BB, BB)], eh_v.at[buf], sem)

        def drain_scatter(buf):
            pltpu.make_async_copy(
                rows_v.at[buf], acc.at[dst2_v.at[buf]], ssems[buf]).wait()

        def process(b, buf):
            sem = sems[buf]
            pltpu.make_async_copy(
                z_hbm.at[src_v.at[pl.ds(0, BB)]],
                rows_v.at[buf], sem).wait()
            pltpu.make_async_copy(
                dstg_hbm.at[wid, 0], dst2_v.at[buf], sem).wait()
            pltpu.make_async_copy(
                eh_hbm.at[pl.ds(0, BB)], eh_v.at[buf], sem).wait()
            for u in range(BB // 16):
                ev = eh_v[buf, pl.ds(u * 16, 16)]
                for l in range(16):
                    a = ev[l]
                    i = u * 16 + l
                    for j in range(H // 16):
                        sl2 = pl.ds(j * 16, 16)
                        rows_v[buf, i, sl2] = rows_v[buf, i, sl2] * a
            pltpu.async_copy(rows_v.at[buf], acc.at[dst2_v.at[buf]],
                             ssems[buf], add=True)

        NR = 3
        for buf in range(NR):
            start(buf, buf)

        def rot(g, carry):
            b0 = g * NR
            for buf in range(NR):
                b = b0 + buf
                process(b, buf)
                @pl.when(b + NR < NB)
                def _():
                    drain_scatter(buf)
                    start(b + NR, buf)
            return carry

        lax.fori_loop(0, NB // NR, rot, 0)
        for r in range(NB % NR):
            process(NB - (NB % NR) + r, r)
        for buf in range(NR):
            drain_scatter(buf)

        plsc.subcore_barrier()
        pltpu.sync_copy(acc.at[pl.ds(s * RPS, RPS), :],
                        part_hbm.at[c, pl.ds(s * RPS, RPS), :])

    return k(z_pad, eh, srcf, dstg, zer2d)


def kernel(x, edge_index, W1, a1_src, a1_dst, b1, W2, a2_src, a2_dst, b2):
    ei = edge_index.astype(jnp.int32)
    x_pad = jnp.pad(x, ((0, NP - N), (0, 0)))
    dstg = ei[1].reshape(NW, NB, BB)
    zer2d = jnp.zeros((RPS, H), jnp.float32)
    zer1d = jnp.zeros((NP,), jnp.float32)

    srcf = ei[0]
    dstf = ei[1]
    z1, elr1, em1 = _tc_in(x_pad, W1, a1_src.reshape(1, H), a1_dst.reshape(1, H))
    eh1, dn1 = _sc_att(elr1[0], elr1[1], em1, srcf, dstf, zer1d)
    p1 = _sc_agg(z1, eh1, srcf, dstg, zer2d)
    z2, elr2, em2 = _tc_mid(p1, dn1, b1.reshape(1, H), W2,
                            a2_src.reshape(1, H), a2_dst.reshape(1, H))
    eh2, dn2 = _sc_att(elr2[0], elr2[1], em2, srcf, dstf, zer1d)
    p2 = _sc_agg(z2, eh2, srcf, dstg, zer2d)
    out = _tc_out(p2, dn2, b2.reshape(1, H))
    return out[:N]


# speedup vs baseline: 1.2474x; 1.2474x over previous
import functools

import jax
import jax.numpy as jnp
from jax import lax
from jax.experimental import pallas as pl
from jax.experimental.pallas import tpu as pltpu
from jax.experimental.pallas import tpu_sc as plsc

N = 10000
NP = 10240
E = 320000
D = 128
H = 128
NC = 2
NS = 16
NW = NC * NS
EPW = E // NW
BB = 80
NB = EPW // BB
RPS = NP // NS
RB = 1024
LEAK = 0.2


def _tc_in(x_pad, W, asrc, adst):
    def body(x_ref, w_ref, as_ref, ad_ref, z_ref, elr_ref, elm_ref):
        z = jnp.dot(x_ref[...], w_ref[...], preferred_element_type=jnp.float32)
        z_ref[...] = z
        el = jnp.sum(z * as_ref[...], axis=1)
        er = jnp.sum(z * ad_ref[...], axis=1)
        elr_ref[...] = jnp.stack([el, er], axis=0)
        @pl.when(pl.program_id(0) == 0)
        def _():
            elm_ref[...] = jnp.full((1, H), -3e38, jnp.float32)
        elm_ref[...] = jnp.maximum(elm_ref[...], jnp.max(el))

    return pl.pallas_call(
        body,
        grid=(NP // RB,),
        in_specs=[
            pl.BlockSpec((RB, D), lambda i: (i, 0)),
            pl.BlockSpec((D, H), lambda i: (0, 0)),
            pl.BlockSpec((1, H), lambda i: (0, 0)),
            pl.BlockSpec((1, H), lambda i: (0, 0)),
        ],
        out_specs=[
            pl.BlockSpec((RB, H), lambda i: (i, 0)),
            pl.BlockSpec((2, RB), lambda i: (0, i)),
            pl.BlockSpec((1, H), lambda i: (0, 0)),
        ],
        out_shape=[
            jax.ShapeDtypeStruct((NP, H), jnp.float32),
            jax.ShapeDtypeStruct((2, NP), jnp.float32),
            jax.ShapeDtypeStruct((1, H), jnp.float32),
        ],
    )(x_pad, W, asrc, adst)


def _tc_mid(part, dens, b, W, asrc, adst):
    def body(p_ref, dn_ref, b_ref, w_ref, as_ref, ad_ref, z_ref, elr_ref,
             elm_ref):
        dt = jnp.sum(dn_ref[...], axis=0) + 1e-38
        agg = (p_ref[0] + p_ref[1]) / dt[:, None] + b_ref[...]
        h = jnp.maximum(agg, 0.0)
        z = jnp.dot(h, w_ref[...], preferred_element_type=jnp.float32)
        z_ref[...] = z
        el = jnp.sum(z * as_ref[...], axis=1)
        er = jnp.sum(z * ad_ref[...], axis=1)
        elr_ref[...] = jnp.stack([el, er], axis=0)
        @pl.when(pl.program_id(0) == 0)
        def _():
            elm_ref[...] = jnp.full((1, H), -3e38, jnp.float32)
        elm_ref[...] = jnp.maximum(elm_ref[...], jnp.max(el))

    return pl.pallas_call(
        body,
        grid=(NP // RB,),
        in_specs=[
            pl.BlockSpec((2, RB, H), lambda i: (0, i, 0)),
            pl.BlockSpec((NW, RB), lambda i: (0, i)),
            pl.BlockSpec((1, H), lambda i: (0, 0)),
            pl.BlockSpec((D, H), lambda i: (0, 0)),
            pl.BlockSpec((1, H), lambda i: (0, 0)),
            pl.BlockSpec((1, H), lambda i: (0, 0)),
        ],
        out_specs=[
            pl.BlockSpec((RB, H), lambda i: (i, 0)),
            pl.BlockSpec((2, RB), lambda i: (0, i)),
            pl.BlockSpec((1, H), lambda i: (0, 0)),
        ],
        out_shape=[
            jax.ShapeDtypeStruct((NP, H), jnp.float32),
            jax.ShapeDtypeStruct((2, NP), jnp.float32),
            jax.ShapeDtypeStruct((1, H), jnp.float32),
        ],
    )(part, dens, b, W, asrc, adst)


def _tc_out(part, dens, b):
    def body(p_ref, dn_ref, b_ref, o_ref):
        dt = jnp.sum(dn_ref[...], axis=0) + 1e-38
        o_ref[...] = (p_ref[0] + p_ref[1]) / dt[:, None] + b_ref[...]

    return pl.pallas_call(
        body,
        grid=(NP // RB,),
        in_specs=[
            pl.BlockSpec((2, RB, H), lambda i: (0, i, 0)),
            pl.BlockSpec((NW, RB), lambda i: (0, i)),
            pl.BlockSpec((1, H), lambda i: (0, 0)),
        ],
        out_specs=pl.BlockSpec((RB, H), lambda i: (i, 0)),
        out_shape=jax.ShapeDtypeStruct((NP, H), jnp.float32),
    )(part, dens, b)


def _sc_att(el, er, elmax, srcf, dstf, zer1d):
    mesh = plsc.VectorSubcoreMesh(core_axis_name="c", subcore_axis_name="s")

    @functools.partial(
        pl.kernel,
        out_type=[
            jax.ShapeDtypeStruct((E,), jnp.float32),
            jax.ShapeDtypeStruct((NW, NP), jnp.float32),
        ],
        mesh=mesh,
        compiler_params=pltpu.CompilerParams(needs_layout_passes=False),
        scratch_types=[
            pltpu.VMEM((EPW,), jnp.int32),
            pltpu.VMEM((EPW,), jnp.int32),
            pltpu.VMEM((NP,), jnp.float32),
            pltpu.VMEM((NP,), jnp.float32),
            pltpu.VMEM((NP,), jnp.float32),
            pltpu.VMEM((1, H), jnp.float32),
            pltpu.VMEM((EPW,), jnp.float32),
        ],
    )
    def k(el_hbm, er_hbm, elm_hbm, srcf_hbm, dstf_hbm, z1_hbm,
          eh_hbm, den_hbm,
          src_v, dst_v, el_v, er_v, den_v, elm_v, eh_v):
        c = lax.axis_index("c")
        s = lax.axis_index("s")
        wid = s * NC + c

        pltpu.sync_copy(srcf_hbm.at[pl.ds(wid * EPW, EPW)], src_v)
        pltpu.sync_copy(dstf_hbm.at[pl.ds(wid * EPW, EPW)], dst_v)
        pltpu.sync_copy(el_hbm, el_v)
        pltpu.sync_copy(er_hbm, er_v)
        pltpu.sync_copy(elm_hbm, elm_v)
        pltpu.sync_copy(z1_hbm, den_v)

        elm = elm_v[0, pl.ds(0, 16)]

        def grp(g, carry):
            sl = pl.ds(g * 16, 16)
            sv = src_v[sl]
            dv = dst_v[sl]
            els = plsc.load_gather(el_v, [sv])
            erd = plsc.load_gather(er_v, [dv])
            e = els + erd
            e = jnp.where(e > 0.0, e, LEAK * e)
            t = erd + elm
            ub = jnp.where(t > 0.0, t, LEAK * t)
            eh = jnp.exp(e - ub)
            eh_v[sl] = eh
            plsc.addupdate_scatter(den_v, [dv], eh)
            return carry

        lax.fori_loop(0, EPW // 16, grp, 0)

        pltpu.sync_copy(eh_v, eh_hbm.at[pl.ds(wid * EPW, EPW)])
        pltpu.sync_copy(den_v, den_hbm.at[wid])

    return k(el, er, elmax, srcf, dstf, zer1d)


def _sc_agg(z_pad, eh, srcf, dstg, zer2d):
    mesh = plsc.VectorSubcoreMesh(core_axis_name="c", subcore_axis_name="s")

    @functools.partial(
        pl.kernel,
        out_type=jax.ShapeDtypeStruct((NC, NP, H), jnp.float32),
        mesh=mesh,
        compiler_params=pltpu.CompilerParams(needs_layout_passes=False),
        scratch_types=[
            pltpu.VMEM((EPW,), jnp.int32),
            pltpu.VMEM((2, BB), jnp.int32),
            pltpu.VMEM((2, BB), jnp.float32),
            pltpu.VMEM((2, BB, H), jnp.float32),
            pltpu.VMEM_SHARED((NP, H), jnp.float32),
            pltpu.SemaphoreType.DMA,
            pltpu.SemaphoreType.DMA,
            pltpu.SemaphoreType.DMA,
            pltpu.SemaphoreType.DMA,
        ],
    )
    def k(z_hbm, eh_hbm, srcf_hbm, dstg_hbm, z2_hbm,
          part_hbm,
          src_v, dst2_v, eh_v, rows_v, acc, sem0, sem1, ssem0, ssem1):
        c = lax.axis_index("c")
        s = lax.axis_index("s")
        wid = s * NC + c

        pltpu.sync_copy(srcf_hbm.at[pl.ds(wid * EPW, EPW)], src_v)
        pltpu.sync_copy(z2_hbm, acc.at[pl.ds(s * RPS, RPS), :])

        plsc.subcore_barrier()

        sems = (sem0, sem1)
        ssems = (ssem0, ssem1)

        def start(b, buf):
            sem = sems[buf]
            pltpu.async_copy(
                z_hbm.at[src_v.at[pl.ds(b * BB, BB)]],
                rows_v.at[buf], sem)
            pltpu.async_copy(dstg_hbm.at[wid, b], dst2_v.at[buf], sem)
            pltpu.async_copy(
                eh_hbm.at[pl.ds(wid * EPW + b * BB, BB)], eh_v.at[buf], sem)

        def drain_scatter(buf):
            pltpu.make_async_copy(
                rows_v.at[buf], acc.at[dst2_v.at[buf]], ssems[buf]).wait()

        def process(b, buf):
            sem = sems[buf]
            pltpu.make_async_copy(
                z_hbm.at[src_v.at[pl.ds(0, BB)]],
                rows_v.at[buf], sem).wait()
            pltpu.make_async_copy(
                dstg_hbm.at[wid, 0], dst2_v.at[buf], sem).wait()
            pltpu.make_async_copy(
                eh_hbm.at[pl.ds(0, BB)], eh_v.at[buf], sem).wait()
            for u in range(BB // 16):
                ev = eh_v[buf, pl.ds(u * 16, 16)]
                for l in range(16):
                    a = ev[l]
                    i = u * 16 + l
                    for j in range(H // 16):
                        sl2 = pl.ds(j * 16, 16)
                        rows_v[buf, i, sl2] = rows_v[buf, i, sl2] * a
            pltpu.async_copy(rows_v.at[buf], acc.at[dst2_v.at[buf]],
                             ssems[buf], add=True)

        NR = 2
        for buf in range(NR):
            start(buf, buf)

        def rot(g, carry):
            b0 = g * NR
            for buf in range(NR):
                b = b0 + buf
                process(b, buf)
                @pl.when(b + NR < NB)
                def _():
                    drain_scatter(buf)
                    start(b + NR, buf)
            return carry

        lax.fori_loop(0, NB // NR, rot, 0)
        for r in range(NB % NR):
            process(NB - (NB % NR) + r, r)
        for buf in range(NR):
            drain_scatter(buf)

        plsc.subcore_barrier()
        pltpu.sync_copy(acc.at[pl.ds(s * RPS, RPS), :],
                        part_hbm.at[c, pl.ds(s * RPS, RPS), :])

    return k(z_pad, eh, srcf, dstg, zer2d)


def kernel(x, edge_index, W1, a1_src, a1_dst, b1, W2, a2_src, a2_dst, b2):
    ei = edge_index.astype(jnp.int32)
    x_pad = jnp.pad(x, ((0, NP - N), (0, 0)))
    dstg = ei[1].reshape(NW, NB, BB)
    zer2d = jnp.zeros((RPS, H), jnp.float32)
    zer1d = jnp.zeros((NP,), jnp.float32)

    srcf = ei[0]
    dstf = ei[1]
    z1, elr1, em1 = _tc_in(x_pad, W1, a1_src.reshape(1, H), a1_dst.reshape(1, H))
    eh1, dn1 = _sc_att(elr1[0], elr1[1], em1, srcf, dstf, zer1d)
    p1 = _sc_agg(z1, eh1, srcf, dstg, zer2d)
    z2, elr2, em2 = _tc_mid(p1, dn1, b1.reshape(1, H), W2,
                            a2_src.reshape(1, H), a2_dst.reshape(1, H))
    eh2, dn2 = _sc_att(elr2[0], elr2[1], em2, srcf, dstf, zer1d)
    p2 = _sc_agg(z2, eh2, srcf, dstg, zer2d)
    out = _tc_out(p2, dn2, b2.reshape(1, H))
    return out[:N]
